# baseline (speedup 1.0000x reference)
import functools

import jax
import jax.numpy as jnp
from jax import lax
from jax.experimental import pallas as pl
from jax.experimental.pallas import tpu as pltpu
from jax.experimental.pallas import tpu_sc as plsc

CHUNKS = (8, 64, 48, 8)
BUFIDX = (0, 1, 2, 0)
BUF_ROWS = (8, 64, 48)


def kernel(x, pos_emb):
    B, S = x.shape
    D = pos_emb.shape[1]

    info = plsc.get_sparse_core_info()
    NC, NS = info.num_cores, info.num_subcores
    NW = NC * NS
    rows_per_w = S // NW
    assert rows_per_w == sum(CHUNKS)
    n_chunks = len(CHUNKS)
    starts = [sum(CHUNKS[:i]) for i in range(n_chunks)]

    mesh = plsc.VectorSubcoreMesh(core_axis_name="c", subcore_axis_name="s")

    @functools.partial(
        pl.kernel,
        out_type=jax.ShapeDtypeStruct((B, S, D), jnp.float32),
        mesh=mesh,
        scratch_types=(
            [pltpu.VMEM((r, D), jnp.float32) for r in BUF_ROWS]
            + [pltpu.SemaphoreType.DMA for _ in range(len(BUF_ROWS) + n_chunks)]
        ),
    )
    def body(pos_hbm, out_hbm, *scratch):
        nb = len(BUF_ROWS)
        bufs = scratch[:nb]
        rsems = scratch[nb:2 * nb]
        wsems = scratch[2 * nb:]
        wid = lax.axis_index("s") * NC + lax.axis_index("c")
        base = wid * rows_per_w

        reads = [None] * n_chunks
        writes = [[] for _ in range(n_chunks)]

        def start_read(c):
            off = base + starts[c]
            cp = pltpu.make_async_copy(
                pos_hbm.at[pl.ds(off, CHUNKS[c])],
                bufs[BUFIDX[c]].at[pl.ds(0, CHUNKS[c])],
                rsems[BUFIDX[c]])
            cp.start()
            reads[c] = cp

        def start_writes(c):
            off = base + starts[c]
            for b in range(B):
                cp = pltpu.make_async_copy(
                    bufs[BUFIDX[c]].at[pl.ds(0, CHUNKS[c])],
                    out_hbm.at[b, pl.ds(off, CHUNKS[c])],
                    wsems[c])
                cp.start()
                writes[c].append(cp)

        start_read(0)
        start_read(1)
        start_read(2)
        reads[0].wait()
        start_writes(0)
        reads[1].wait()
        start_writes(1)
        for cp in writes[0]:
            cp.wait()
        start_read(3)
        reads[2].wait()
        start_writes(2)
        reads[3].wait()
        start_writes(3)
        for c in (1, 2, 3):
            for cp in writes[c]:
                cp.wait()

    return body(pos_emb)


# speedup vs baseline: 1.0165x; 1.0165x over previous
import functools

import jax
import jax.numpy as jnp
from jax import lax
from jax.experimental import pallas as pl
from jax.experimental.pallas import tpu as pltpu
from jax.experimental.pallas import tpu_sc as plsc

STREAM_CHUNKS = (48, 40)
SP_ROWS = 40


def kernel(x, pos_emb):
    B, S = x.shape
    D = pos_emb.shape[1]

    info = plsc.get_sparse_core_info()
    NC, NS = info.num_cores, info.num_subcores
    NW = NC * NS
    rows_per_w = S // NW
    assert rows_per_w == sum(STREAM_CHUNKS) + SP_ROWS
    starts = [0, STREAM_CHUNKS[0]]
    sp_start = sum(STREAM_CHUNKS)

    mesh = plsc.VectorSubcoreMesh(core_axis_name="c", subcore_axis_name="s")

    @functools.partial(
        pl.kernel,
        out_type=jax.ShapeDtypeStruct((B, S, D), jnp.float32),
        mesh=mesh,
        scratch_types=[
            pltpu.VMEM((STREAM_CHUNKS[0], D), jnp.float32),
            pltpu.VMEM((STREAM_CHUNKS[1], D), jnp.float32),
            pltpu.VMEM_SHARED((NS * SP_ROWS, D), jnp.float32),
            pltpu.SemaphoreType.DMA,
            pltpu.SemaphoreType.DMA,
            pltpu.SemaphoreType.DMA,
            pltpu.SemaphoreType.DMA,
            pltpu.SemaphoreType.DMA,
            pltpu.SemaphoreType.DMA,
        ],
    )
    def body(pos_hbm, out_hbm, tbuf0, tbuf1, spbuf,
             rsem0, rsem1, rsem_sp, wsem0, wsem1, wsem_sp):
        sid = lax.axis_index("s")
        wid = sid * NC + lax.axis_index("c")
        base = wid * rows_per_w
        tbufs = (tbuf0, tbuf1)
        rsems = (rsem0, rsem1)
        wsems = (wsem0, wsem1)
        spslice = spbuf.at[pl.ds(sid * SP_ROWS, SP_ROWS)]

        sp_read = pltpu.make_async_copy(
            pos_hbm.at[pl.ds(base + sp_start, SP_ROWS)], spslice, rsem_sp)
        sp_read.start()
        reads = []
        for c in range(2):
            cp = pltpu.make_async_copy(
                pos_hbm.at[pl.ds(base + starts[c], STREAM_CHUNKS[c])],
                tbufs[c], rsems[c])
            cp.start()
            reads.append(cp)

        writes = []
        reads[0].wait()
        for b in range(B):
            cp = pltpu.make_async_copy(
                tbufs[0], out_hbm.at[b, pl.ds(base + starts[0], STREAM_CHUNKS[0])],
                wsems[0])
            cp.start()
            writes.append(cp)
        sp_read.wait()
        for b in range(B):
            cp = pltpu.make_async_copy(
                spslice, out_hbm.at[b, pl.ds(base + sp_start, SP_ROWS)],
                wsem_sp)
            cp.start()
            writes.append(cp)
        reads[1].wait()
        for b in range(B):
            cp = pltpu.make_async_copy(
                tbufs[1], out_hbm.at[b, pl.ds(base + starts[1], STREAM_CHUNKS[1])],
                wsems[1])
            cp.start()
            writes.append(cp)
        for cp in writes:
            cp.wait()

    return body(pos_emb)


# speedup vs baseline: 1.0176x; 1.0010x over previous
import functools

import jax
import jax.numpy as jnp
from jax import lax
from jax.experimental import pallas as pl
from jax.experimental.pallas import tpu as pltpu
from jax.experimental.pallas import tpu_sc as plsc

STREAM_CHUNKS = (40, 32)
SP_ROWS = 56


def kernel(x, pos_emb):
    B, S = x.shape
    D = pos_emb.shape[1]

    info = plsc.get_sparse_core_info()
    NC, NS = info.num_cores, info.num_subcores
    NW = NC * NS
    rows_per_w = S // NW
    assert rows_per_w == sum(STREAM_CHUNKS) + SP_ROWS
    starts = [0, STREAM_CHUNKS[0]]
    sp_start = sum(STREAM_CHUNKS)

    mesh = plsc.VectorSubcoreMesh(core_axis_name="c", subcore_axis_name="s")

    @functools.partial(
        pl.kernel,
        out_type=jax.ShapeDtypeStruct((B, S, D), jnp.float32),
        mesh=mesh,
        scratch_types=[
            pltpu.VMEM((STREAM_CHUNKS[0], D), jnp.float32),
            pltpu.VMEM((STREAM_CHUNKS[1], D), jnp.float32),
            pltpu.VMEM_SHARED((NS * SP_ROWS, D), jnp.float32),
            pltpu.SemaphoreType.DMA,
            pltpu.SemaphoreType.DMA,
            pltpu.SemaphoreType.DMA,
            pltpu.SemaphoreType.DMA,
            pltpu.SemaphoreType.DMA,
            pltpu.SemaphoreType.DMA,
        ],
    )
    def body(pos_hbm, out_hbm, tbuf0, tbuf1, spbuf,
             rsem0, rsem1, rsem_sp, wsem0, wsem1, wsem_sp):
        sid = lax.axis_index("s")
        wid = sid * NC + lax.axis_index("c")
        base = wid * rows_per_w
        tbufs = (tbuf0, tbuf1)
        rsems = (rsem0, rsem1)
        wsems = (wsem0, wsem1)
        spslice = spbuf.at[pl.ds(sid * SP_ROWS, SP_ROWS)]

        sp_read = pltpu.make_async_copy(
            pos_hbm.at[pl.ds(base + sp_start, SP_ROWS)], spslice, rsem_sp)
        sp_read.start()
        reads = []
        for c in range(2):
            cp = pltpu.make_async_copy(
                pos_hbm.at[pl.ds(base + starts[c], STREAM_CHUNKS[c])],
                tbufs[c], rsems[c])
            cp.start()
            reads.append(cp)

        writes = []
        reads[0].wait()
        for b in range(B):
            cp = pltpu.make_async_copy(
                tbufs[0], out_hbm.at[b, pl.ds(base + starts[0], STREAM_CHUNKS[0])],
                wsems[0])
            cp.start()
            writes.append(cp)
        sp_read.wait()
        for b in range(B):
            cp = pltpu.make_async_copy(
                spslice, out_hbm.at[b, pl.ds(base + sp_start, SP_ROWS)],
                wsem_sp)
            cp.start()
            writes.append(cp)
        reads[1].wait()
        for b in range(B):
            cp = pltpu.make_async_copy(
                tbufs[1], out_hbm.at[b, pl.ds(base + starts[1], STREAM_CHUNKS[1])],
                wsems[1])
            cp.start()
            writes.append(cp)
        for cp in writes:
            cp.wait()

    return body(pos_emb)


# speedup vs baseline: 1.0227x; 1.0050x over previous
import functools

import jax
import jax.numpy as jnp
from jax import lax
from jax.experimental import pallas as pl
from jax.experimental.pallas import tpu as pltpu
from jax.experimental.pallas import tpu_sc as plsc

STREAM_CHUNKS = (72,)
SP_ROWS = 56


def kernel(x, pos_emb):
    B, S = x.shape
    D = pos_emb.shape[1]

    info = plsc.get_sparse_core_info()
    NC, NS = info.num_cores, info.num_subcores
    NW = NC * NS
    rows_per_w = S // NW
    assert rows_per_w == sum(STREAM_CHUNKS) + SP_ROWS
    starts = [0]
    sp_start = sum(STREAM_CHUNKS)

    mesh = plsc.VectorSubcoreMesh(core_axis_name="c", subcore_axis_name="s")

    @functools.partial(
        pl.kernel,
        out_type=jax.ShapeDtypeStruct((B, S, D), jnp.float32),
        mesh=mesh,
        scratch_types=[
            pltpu.VMEM((STREAM_CHUNKS[0], D), jnp.float32),
            pltpu.VMEM_SHARED((NS * SP_ROWS, D), jnp.float32),
            pltpu.SemaphoreType.DMA,
            pltpu.SemaphoreType.DMA,
            pltpu.SemaphoreType.DMA,
            pltpu.SemaphoreType.DMA,
        ],
    )
    def body(pos_hbm, out_hbm, tbuf0, spbuf,
             rsem0, rsem_sp, wsem0, wsem_sp):
        sid = lax.axis_index("s")
        wid = sid * NC + lax.axis_index("c")
        base = wid * rows_per_w
        tbufs = (tbuf0,)
        rsems = (rsem0,)
        wsems = (wsem0,)
        spslice = spbuf.at[pl.ds(sid * SP_ROWS, SP_ROWS)]

        sp_read = pltpu.make_async_copy(
            pos_hbm.at[pl.ds(base + sp_start, SP_ROWS)], spslice, rsem_sp)
        sp_read.start()
        reads = []
        for c in range(len(STREAM_CHUNKS)):
            cp = pltpu.make_async_copy(
                pos_hbm.at[pl.ds(base + starts[c], STREAM_CHUNKS[c])],
                tbufs[c], rsems[c])
            cp.start()
            reads.append(cp)

        writes = []
        reads[0].wait()
        for b in range(B):
            cp = pltpu.make_async_copy(
                tbufs[0], out_hbm.at[b, pl.ds(base + starts[0], STREAM_CHUNKS[0])],
                wsems[0])
            cp.start()
            writes.append(cp)
        sp_read.wait()
        for b in range(B):
            cp = pltpu.make_async_copy(
                spslice, out_hbm.at[b, pl.ds(base + sp_start, SP_ROWS)],
                wsem_sp)
            cp.start()
            writes.append(cp)
        for cp in writes:
            cp.wait()

    return body(pos_emb)


# speedup vs baseline: 1.0312x; 1.0084x over previous
import functools

import jax
import jax.numpy as jnp
from jax import lax
from jax.experimental import pallas as pl
from jax.experimental.pallas import tpu as pltpu
from jax.experimental.pallas import tpu_sc as plsc

STREAM_CHUNKS = (120,)
SP_ROWS = 8


def kernel(x, pos_emb):
    B, S = x.shape
    D = pos_emb.shape[1]

    info = plsc.get_sparse_core_info()
    NC, NS = info.num_cores, info.num_subcores
    NW = NC * NS
    rows_per_w = S // NW
    assert rows_per_w == sum(STREAM_CHUNKS) + SP_ROWS
    starts = [0]
    sp_start = sum(STREAM_CHUNKS)

    mesh = plsc.VectorSubcoreMesh(core_axis_name="c", subcore_axis_name="s")

    @functools.partial(
        pl.kernel,
        out_type=jax.ShapeDtypeStruct((B, S, D), jnp.float32),
        mesh=mesh,
        scratch_types=[
            pltpu.VMEM((STREAM_CHUNKS[0], D), jnp.float32),
            pltpu.VMEM_SHARED((NS * SP_ROWS, D), jnp.float32),
            pltpu.SemaphoreType.DMA,
            pltpu.SemaphoreType.DMA,
            pltpu.SemaphoreType.DMA,
            pltpu.SemaphoreType.DMA,
        ],
    )
    def body(pos_hbm, out_hbm, tbuf0, spbuf,
             rsem0, rsem_sp, wsem0, wsem_sp):
        sid = lax.axis_index("s")
        wid = sid * NC + lax.axis_index("c")
        base = wid * rows_per_w
        tbufs = (tbuf0,)
        rsems = (rsem0,)
        wsems = (wsem0,)
        spslice = spbuf.at[pl.ds(sid * SP_ROWS, SP_ROWS)]

        sp_read = pltpu.make_async_copy(
            pos_hbm.at[pl.ds(base + sp_start, SP_ROWS)], spslice, rsem_sp)
        sp_read.start()
        reads = []
        for c in range(len(STREAM_CHUNKS)):
            cp = pltpu.make_async_copy(
                pos_hbm.at[pl.ds(base + starts[c], STREAM_CHUNKS[c])],
                tbufs[c], rsems[c])
            cp.start()
            reads.append(cp)

        writes = []
        reads[0].wait()
        for b in range(B):
            cp = pltpu.make_async_copy(
                tbufs[0], out_hbm.at[b, pl.ds(base + starts[0], STREAM_CHUNKS[0])],
                wsems[0])
            cp.start()
            writes.append(cp)
        sp_read.wait()
        for b in range(B):
            cp = pltpu.make_async_copy(
                spslice, out_hbm.at[b, pl.ds(base + sp_start, SP_ROWS)],
                wsem_sp)
            cp.start()
            writes.append(cp)
        for cp in writes:
            cp.wait()

    return body(pos_emb)
